# baseline (speedup 1.0000x reference)
import functools

import jax
import jax.numpy as jnp
from jax import lax
from jax.experimental import pallas as pl
from jax.experimental.pallas import tpu as pltpu
from jax.experimental.pallas import tpu_sc as plsc

_NC = 2
_NS = 16
_NW = _NC * _NS
_L = 16
_C = 16
_NBUF = 5
_LOOKAHEAD = 3


def kernel(input_ids, word_embeddings, position_embeddings):
    B, S = input_ids.shape
    _, H = word_embeddings.shape
    W = S // _NW
    n_chunks = W // _C
    ids = input_ids.astype(jnp.int32)

    mesh = plsc.VectorSubcoreMesh(core_axis_name="c", subcore_axis_name="s")

    @functools.partial(
        pl.kernel,
        mesh=mesh,
        out_type=jax.ShapeDtypeStruct((B * S, H), jnp.float32),
        scratch_types=(
            [pltpu.VMEM((B * W,), jnp.int32)]
            + [pltpu.VMEM((_C, H), jnp.float32) for _ in range(_NBUF + 2)]
            + [pltpu.SemaphoreType.DMA for _ in range(2 * _NBUF + 3)]
        ),
    )
    def embed(ids_hbm, we_hbm, pe_hbm, out_hbm, idx_v, *bufs_and_sems):
        rbufs = bufs_and_sems[:_NBUF]
        pbufs = bufs_and_sems[_NBUF:_NBUF + 2]
        sems = bufs_and_sems[_NBUF + 2:]
        gsems = sems[:_NBUF]
        wsems = sems[_NBUF:2 * _NBUF]
        psems = sems[2 * _NBUF:2 * _NBUF + 2]
        isem = sems[2 * _NBUF + 2]

        wid = lax.axis_index("s") * _NC + lax.axis_index("c")
        s_base = wid * W

        i_cps = [
            pltpu.async_copy(ids_hbm.at[b].at[pl.ds(s_base, W)],
                             idx_v.at[pl.ds(b * W, W)], isem)
            for b in range(B)
        ]
        n_items = n_chunks * B

        def item_params(i):
            return i // B, i % B

        def fire_gather(i):
            sc, b = item_params(i)
            return pltpu.async_copy(
                we_hbm.at[idx_v.at[pl.ds(b * W + sc * _C, _C)]],
                rbufs[i % _NBUF], gsems[i % _NBUF])

        def fire_pos(sc):
            return pltpu.async_copy(
                pe_hbm.at[pl.ds(s_base + sc * _C, _C)],
                pbufs[sc % 2], psems[sc % 2])

        def add_pos(buf, pbuf):
            @pl.loop(0, _C)
            def _row(r):
                @plsc.parallel_loop(0, H, step=_L, unroll=16)
                def _col(c):
                    buf.at[r, pl.ds(c, _L)][...] = (
                        buf.at[r, pl.ds(c, _L)][...]
                        + pbuf.at[r, pl.ds(c, _L)][...])

        p_cps = {0: fire_pos(0)}
        for cp in i_cps:
            cp.wait()
        g_cps = {i: fire_gather(i) for i in range(min(_LOOKAHEAD, n_items))}
        w_cps = {}
        w_waited = set()
        for i in range(n_items):
            k = i % _NBUF
            sc, b = item_params(i)
            if b == 0:
                p_cps[sc].wait()
                if sc + 1 < n_chunks:
                    p_cps[sc + 1] = fire_pos(sc + 1)
            j = i + _LOOKAHEAD
            if j < n_items:
                if j - _NBUF in w_cps:
                    w_cps[j - _NBUF].wait()
                    w_waited.add(j - _NBUF)
                g_cps[j] = fire_gather(j)
            g_cps[i].wait()
            add_pos(rbufs[k], pbufs[sc % 2])
            w_cps[i] = pltpu.async_copy(
                rbufs[k], out_hbm.at[pl.ds(b * S + s_base + sc * _C, _C)],
                wsems[k])
        for i in range(n_items):
            if i not in w_waited:
                w_cps[i].wait()

    out = embed(ids, word_embeddings, position_embeddings)
    return out.reshape(B, S, H)


# speedup vs baseline: 1.0339x; 1.0339x over previous
import functools

import jax
import jax.numpy as jnp
from jax import lax
from jax.experimental import pallas as pl
from jax.experimental.pallas import tpu as pltpu
from jax.experimental.pallas import tpu_sc as plsc

_NC = 2
_NS = 16
_NW = _NC * _NS
_L = 16
_C = 16
_NBUF = 5
_LOOKAHEAD = 2


def kernel(input_ids, word_embeddings, position_embeddings):
    B, S = input_ids.shape
    _, H = word_embeddings.shape
    W = S // _NW
    n_chunks = W // _C
    ids = input_ids.astype(jnp.int32)

    mesh = plsc.VectorSubcoreMesh(core_axis_name="c", subcore_axis_name="s")

    @functools.partial(
        pl.kernel,
        mesh=mesh,
        out_type=jax.ShapeDtypeStruct((B * S, H), jnp.float32),
        scratch_types=(
            [pltpu.VMEM((B * W,), jnp.int32)]
            + [pltpu.VMEM((_C, H), jnp.float32) for _ in range(_NBUF + 2)]
            + [pltpu.SemaphoreType.DMA for _ in range(2 * _NBUF + 3)]
        ),
    )
    def embed(ids_hbm, we_hbm, pe_hbm, out_hbm, idx_v, *bufs_and_sems):
        rbufs = bufs_and_sems[:_NBUF]
        pbufs = bufs_and_sems[_NBUF:_NBUF + 2]
        sems = bufs_and_sems[_NBUF + 2:]
        gsems = sems[:_NBUF]
        wsems = sems[_NBUF:2 * _NBUF]
        psems = sems[2 * _NBUF:2 * _NBUF + 2]
        isem = sems[2 * _NBUF + 2]

        wid = lax.axis_index("s") * _NC + lax.axis_index("c")
        s_base = wid * W

        i_cps = [
            pltpu.async_copy(ids_hbm.at[b].at[pl.ds(s_base, W)],
                             idx_v.at[pl.ds(b * W, W)], isem)
            for b in range(B)
        ]
        n_items = n_chunks * B

        def item_params(i):
            return i // B, i % B

        def fire_gather(i):
            sc, b = item_params(i)
            return pltpu.async_copy(
                we_hbm.at[idx_v.at[pl.ds(b * W + sc * _C, _C)]],
                rbufs[i % _NBUF], gsems[i % _NBUF])

        def fire_pos(sc):
            return pltpu.async_copy(
                pe_hbm.at[pl.ds(s_base + sc * _C, _C)],
                pbufs[sc % 2], psems[sc % 2])

        def add_pos(buf, pbuf):
            @pl.loop(0, _C)
            def _row(r):
                @plsc.parallel_loop(0, H, step=_L, unroll=8)
                def _col(c):
                    buf.at[r, pl.ds(c, _L)][...] = (
                        buf.at[r, pl.ds(c, _L)][...]
                        + pbuf.at[r, pl.ds(c, _L)][...])

        p_cps = {0: fire_pos(0)}
        for cp in i_cps:
            cp.wait()
        g_cps = {i: fire_gather(i) for i in range(min(_LOOKAHEAD, n_items))}
        w_cps = {}
        w_waited = set()
        for i in range(n_items):
            k = i % _NBUF
            sc, b = item_params(i)
            if b == 0:
                p_cps[sc].wait()
                if sc + 1 < n_chunks:
                    p_cps[sc + 1] = fire_pos(sc + 1)
            j = i + _LOOKAHEAD
            if j < n_items:
                if j - _NBUF in w_cps:
                    w_cps[j - _NBUF].wait()
                    w_waited.add(j - _NBUF)
                g_cps[j] = fire_gather(j)
            g_cps[i].wait()
            add_pos(rbufs[k], pbufs[sc % 2])
            w_cps[i] = pltpu.async_copy(
                rbufs[k], out_hbm.at[pl.ds(b * S + s_base + sc * _C, _C)],
                wsems[k])
        for i in range(n_items):
            if i not in w_waited:
                w_cps[i].wait()

    out = embed(ids, word_embeddings, position_embeddings)
    return out.reshape(B, S, H)
